# Initial kernel scaffold; baseline (speedup 1.0000x reference)
import jax
import jax.numpy as jnp
from jax.experimental import pallas as pl


def kernel(user_id, user_age, country, movie_id, movie_genres, user_table, age_table, country_table, movie_table, genre_table, W1, b1, W2, b2, W3, b3):
    raise NotImplementedError("write your pallas kernel here")


# speedup vs baseline: 9.8219x; 9.8219x over previous
import functools

import jax
import jax.numpy as jnp
from jax import lax
from jax.experimental import pallas as pl
from jax.experimental.pallas import tpu as pltpu
from jax.experimental.pallas import tpu_sc as plsc

B = 16384
D = 32
NC = 2
NS = 16
NW = NC * NS
B_PER_W = B // NW
CHUNK = 128
N_CHUNKS = B_PER_W // CHUNK

GVOC = 21
GPAD = 32
N_GENRES = 19

BLK = 4096


def _sc_gather_body(uid, aid, cid, mid, ut, at_, ct, mt,
                    ou, oa, oc, om, idx_v, rows_v, sem):
    wid = lax.axis_index("s") * NC + lax.axis_index("c")
    base = wid * B_PER_W
    for idx_hbm, tab, out in ((uid, ut, ou), (aid, at_, oa),
                              (cid, ct, oc), (mid, mt, om)):
        for c in range(N_CHUNKS):
            off = base + c * CHUNK
            pltpu.sync_copy(idx_hbm.at[pl.ds(off, CHUNK)], idx_v)
            pltpu.async_copy(tab.at[idx_v], rows_v, sem).wait()
            pltpu.sync_copy(rows_v, out.at[pl.ds(off, CHUNK)])


def _sc_gather(uid, aid, cid, mid, ut, at_, ct, mt):
    mesh = plsc.VectorSubcoreMesh(core_axis_name="c", subcore_axis_name="s")
    f = functools.partial(
        pl.kernel,
        mesh=mesh,
        compiler_params=pltpu.CompilerParams(use_tc_tiling_on_sc=False),
        out_type=[jax.ShapeDtypeStruct((B, D), jnp.float32)] * 4,
        scratch_types=[
            pltpu.VMEM((CHUNK,), jnp.int32),
            pltpu.VMEM((CHUNK, D), jnp.float32),
            pltpu.SemaphoreType.DMA,
        ],
    )(_sc_gather_body)
    return f(uid, aid, cid, mid, ut, at_, ct, mt)


def _mlp_body(ue, ae, ce, me, gen, gt, W1, b1, W2, b2, W3, b3, out):
    g = gen[...]
    vocab = lax.broadcasted_iota(jnp.int32, (BLK, GPAD), 1)
    counts = jnp.zeros((BLK, GPAD), jnp.float32)
    for t in range(N_GENRES):
        counts += (vocab == g[:, t:t + 1]).astype(jnp.float32)
    ge = jnp.dot(counts, gt[...], preferred_element_type=jnp.float32)
    ge = ge * jnp.float32(1.0 / N_GENRES)

    w1 = W1[...]
    h = jnp.dot(ue[...], w1[0:D, :], preferred_element_type=jnp.float32)
    h += jnp.dot(ae[...], w1[D:2 * D, :], preferred_element_type=jnp.float32)
    h += jnp.dot(ce[...], w1[2 * D:3 * D, :], preferred_element_type=jnp.float32)
    h += jnp.dot(me[...], w1[3 * D:4 * D, :], preferred_element_type=jnp.float32)
    h += jnp.dot(ge, w1[4 * D:5 * D, :], preferred_element_type=jnp.float32)
    h = jnp.maximum(h + b1[...], 0.0)
    h2 = jnp.maximum(jnp.dot(h, W2[...], preferred_element_type=jnp.float32) + b2[...], 0.0)
    out[...] = jnp.dot(h2, W3[...], preferred_element_type=jnp.float32) + b3[...]


def _mlp(ue, ae, ce, me, gen, gt, W1, b1, W2, b2, W3, b3):
    grid = (B // BLK,)
    bspec = lambda shape: pl.BlockSpec(shape, lambda i: (i, 0))
    full = lambda shape: pl.BlockSpec(shape, lambda i: (0, 0))
    return pl.pallas_call(
        _mlp_body,
        grid=grid,
        in_specs=[
            bspec((BLK, D)), bspec((BLK, D)), bspec((BLK, D)), bspec((BLK, D)),
            bspec((BLK, N_GENRES)),
            full((GPAD, D)),
            full((5 * D, 256)), full((1, 256)),
            full((256, 64)), full((1, 64)),
            full((64, 1)), full((1, 1)),
        ],
        out_specs=bspec((BLK, 1)),
        out_shape=jax.ShapeDtypeStruct((B, 1), jnp.float32),
    )(ue, ae, ce, me, gen, gt, W1, b1, W2, b2, W3, b3)


def kernel(user_id, user_age, country, movie_id, movie_genres,
           user_table, age_table, country_table, movie_table, genre_table,
           W1, b1, W2, b2, W3, b3):
    uid = user_id.reshape(B)
    aid = user_age.reshape(B)
    cid = country.reshape(B)
    mid = movie_id.reshape(B)
    ue, ae, ce, me = _sc_gather(uid, aid, cid, mid,
                                user_table, age_table, country_table,
                                movie_table)
    gt_pad = jnp.pad(genre_table, ((0, GPAD - GVOC), (0, 0)))
    out = _mlp(ue, ae, ce, me, movie_genres, gt_pad,
               W1, b1.reshape(1, 256), W2, b2.reshape(1, 64),
               W3, b3.reshape(1, 1))
    return out.reshape(B, 1, 1)


# speedup vs baseline: 10.6944x; 1.0888x over previous
import functools

import jax
import jax.numpy as jnp
from jax import lax
from jax.experimental import pallas as pl
from jax.experimental.pallas import tpu as pltpu
from jax.experimental.pallas import tpu_sc as plsc

B = 16384
D = 32
NC = 2
NS = 16
NW = NC * NS
B_PER_W = B // NW
CHUNK = 128
N_CHUNKS = B_PER_W // CHUNK

GVOC = 21
GPAD = 32
N_GENRES = 19

BLK = 4096


def _sc_gather_body(uid, aid, cid, mid, ut, at_, ct, mt,
                    ou, oa, oc, om,
                    iu, ia, ic, im, ru, ra, rc, rm, isem, gsem, wsem):
    wid = lax.axis_index("s") * NC + lax.axis_index("c")
    base = wid * B_PER_W
    base2d = wid * N_CHUNKS
    feats = ((uid, ut, ou, iu, ru), (aid, at_, oa, ia, ra),
             (cid, ct, oc, ic, rc), (mid, mt, om, im, rm))
    loads = [pltpu.async_copy(idx2d.at[pl.ds(base2d, N_CHUNKS)], iv, isem)
             for idx2d, _, _, iv, _ in feats]
    for h in loads:
        h.wait()
    gathers = []
    for _, tab, _, iv, rv in feats:
        for c in range(N_CHUNKS):
            gathers.append(pltpu.async_copy(
                tab.at[iv.at[c]], rv.at[pl.ds(c * CHUNK, CHUNK)], gsem))
    for h in gathers:
        h.wait()
    writes = [pltpu.async_copy(rv, out.at[pl.ds(base, B_PER_W)], wsem)
              for _, _, out, _, rv in feats]
    for h in writes:
        h.wait()


def _sc_gather(uid, aid, cid, mid, ut, at_, ct, mt):
    mesh = plsc.VectorSubcoreMesh(core_axis_name="c", subcore_axis_name="s")
    f = functools.partial(
        pl.kernel,
        mesh=mesh,
        compiler_params=pltpu.CompilerParams(use_tc_tiling_on_sc=False),
        out_type=[jax.ShapeDtypeStruct((B, D), jnp.float32)] * 4,
        scratch_types=[pltpu.VMEM((N_CHUNKS, CHUNK), jnp.int32)] * 4
                      + [pltpu.VMEM((B_PER_W, D), jnp.float32)] * 4
                      + [pltpu.SemaphoreType.DMA] * 3,
    )(_sc_gather_body)
    return f(uid.reshape(B // CHUNK, CHUNK), aid.reshape(B // CHUNK, CHUNK),
             cid.reshape(B // CHUNK, CHUNK), mid.reshape(B // CHUNK, CHUNK),
             ut, at_, ct, mt)


def _mlp_body(ue, ae, ce, me, gen, gt, W1, b1, W2, b2, W3, b3, out):
    g = gen[...]
    vocab = lax.broadcasted_iota(jnp.int32, (BLK, GPAD), 1)
    counts = jnp.zeros((BLK, GPAD), jnp.float32)
    for t in range(N_GENRES):
        counts += (vocab == g[:, t:t + 1]).astype(jnp.float32)
    ge = jnp.dot(counts, gt[...], preferred_element_type=jnp.float32)
    ge = ge * jnp.float32(1.0 / N_GENRES)

    w1 = W1[...]
    h = jnp.dot(ue[...], w1[0:D, :], preferred_element_type=jnp.float32)
    h += jnp.dot(ae[...], w1[D:2 * D, :], preferred_element_type=jnp.float32)
    h += jnp.dot(ce[...], w1[2 * D:3 * D, :], preferred_element_type=jnp.float32)
    h += jnp.dot(me[...], w1[3 * D:4 * D, :], preferred_element_type=jnp.float32)
    h += jnp.dot(ge, w1[4 * D:5 * D, :], preferred_element_type=jnp.float32)
    h = jnp.maximum(h + b1[...], 0.0)
    h2 = jnp.maximum(jnp.dot(h, W2[...], preferred_element_type=jnp.float32) + b2[...], 0.0)
    out[...] = jnp.dot(h2, W3[...], preferred_element_type=jnp.float32) + b3[...]


def _mlp(ue, ae, ce, me, gen, gt, W1, b1, W2, b2, W3, b3):
    grid = (B // BLK,)
    bspec = lambda shape: pl.BlockSpec(shape, lambda i: (i, 0))
    full = lambda shape: pl.BlockSpec(shape, lambda i: (0, 0))
    return pl.pallas_call(
        _mlp_body,
        grid=grid,
        in_specs=[
            bspec((BLK, D)), bspec((BLK, D)), bspec((BLK, D)), bspec((BLK, D)),
            bspec((BLK, N_GENRES)),
            full((GPAD, D)),
            full((5 * D, 256)), full((1, 256)),
            full((256, 64)), full((1, 64)),
            full((64, 1)), full((1, 1)),
        ],
        out_specs=bspec((BLK, 1)),
        out_shape=jax.ShapeDtypeStruct((B, 1), jnp.float32),
    )(ue, ae, ce, me, gen, gt, W1, b1, W2, b2, W3, b3)


def kernel(user_id, user_age, country, movie_id, movie_genres,
           user_table, age_table, country_table, movie_table, genre_table,
           W1, b1, W2, b2, W3, b3):
    uid = user_id.reshape(B)
    aid = user_age.reshape(B)
    cid = country.reshape(B)
    mid = movie_id.reshape(B)
    ue, ae, ce, me = _sc_gather(uid, aid, cid, mid,
                                user_table, age_table, country_table,
                                movie_table)
    gt_pad = jnp.pad(genre_table, ((0, GPAD - GVOC), (0, 0)))
    out = _mlp(ue, ae, ce, me, movie_genres, gt_pad,
               W1, b1.reshape(1, 256), W2, b2.reshape(1, 64),
               W3, b3.reshape(1, 1))
    return out.reshape(B, 1, 1)


# speedup vs baseline: 13.2142x; 1.2356x over previous
import functools

import jax
import jax.numpy as jnp
from jax import lax
from jax.experimental import pallas as pl
from jax.experimental.pallas import tpu as pltpu
from jax.experimental.pallas import tpu_sc as plsc

B = 16384
D = 32
NC = 2
NS = 16
NW = NC * NS
B_PER_W = B // NW
CHUNK = 128
N_CHUNKS = B_PER_W // CHUNK

GVOC = 21
GPAD = 32
N_GENRES = 19

BLK = 4096


def _sc_gather_body(uid, aid, cid, mid, ut, at_, ct, mt, out,
                    iu, ia, ic, im, ru, ra, rc, rm, isem, gsem, wsem):
    wid = lax.axis_index("s") * NC + lax.axis_index("c")
    base = wid * B_PER_W
    feats = ((uid, ut, iu, ru), (aid, at_, ia, ra),
             (cid, ct, ic, rc), (mid, mt, im, rm))
    loads = [pltpu.async_copy(idx.at[pl.ds(base, B_PER_W)], iv, isem)
             for idx, _, iv, _ in feats]
    for h in loads:
        h.wait()
    gathers = []
    for _, tab, iv, rv in feats:
        for c in range(N_CHUNKS):
            gathers.append(pltpu.async_copy(
                tab.at[iv.at[pl.ds(c * CHUNK, CHUNK)]],
                rv.at[pl.ds(c * CHUNK, CHUNK)], gsem))
    for h in gathers:
        h.wait()
    writes = [pltpu.async_copy(rv, out.at[pl.ds(base, B_PER_W),
                                          pl.ds(f * D, D)], wsem)
              for f, (_, _, _, rv) in enumerate(feats)]
    for h in writes:
        h.wait()


def _sc_gather(uid, aid, cid, mid, ut, at_, ct, mt):
    mesh = plsc.VectorSubcoreMesh(core_axis_name="c", subcore_axis_name="s")
    f = functools.partial(
        pl.kernel,
        mesh=mesh,
        compiler_params=pltpu.CompilerParams(use_tc_tiling_on_sc=False),
        out_type=jax.ShapeDtypeStruct((B, 4 * D), jnp.float32),
        scratch_types=[pltpu.VMEM((B_PER_W,), jnp.int32)] * 4
                      + [pltpu.VMEM((B_PER_W, D), jnp.float32)] * 4
                      + [pltpu.SemaphoreType.DMA] * 3,
    )(_sc_gather_body)
    return f(uid, aid, cid, mid, ut, at_, ct, mt)


def _mlp_body(e4, gen, gt, W1, b1, W2, b2, W3, b3, out):
    g = gen[...]
    vocab = lax.broadcasted_iota(jnp.int32, (BLK, GPAD), 1)
    counts = jnp.zeros((BLK, GPAD), jnp.float32)
    for t in range(N_GENRES):
        counts += (vocab == g[:, t:t + 1]).astype(jnp.float32)
    ge = jnp.dot(counts, gt[...], preferred_element_type=jnp.float32)
    ge = ge * jnp.float32(1.0 / N_GENRES)

    w1 = W1[...]
    h = jnp.dot(e4[...], w1[0:4 * D, :], preferred_element_type=jnp.float32)
    h += jnp.dot(ge, w1[4 * D:5 * D, :], preferred_element_type=jnp.float32)
    h = jnp.maximum(h + b1[...], 0.0)
    h2 = jnp.maximum(jnp.dot(h, W2[...], preferred_element_type=jnp.float32) + b2[...], 0.0)
    out[...] = jnp.dot(h2, W3[...], preferred_element_type=jnp.float32) + b3[...]


def _mlp(e4, gen, gt, W1, b1, W2, b2, W3, b3):
    grid = (B // BLK,)
    bspec = lambda shape: pl.BlockSpec(shape, lambda i: (i, 0))
    full = lambda shape: pl.BlockSpec(shape, lambda i: (0, 0))
    return pl.pallas_call(
        _mlp_body,
        grid=grid,
        in_specs=[
            bspec((BLK, 4 * D)),
            bspec((BLK, N_GENRES)),
            full((GPAD, D)),
            full((5 * D, 256)), full((1, 256)),
            full((256, 64)), full((1, 64)),
            full((64, 1)), full((1, 1)),
        ],
        out_specs=bspec((BLK, 1)),
        out_shape=jax.ShapeDtypeStruct((B, 1), jnp.float32),
    )(e4, gen, gt, W1, b1, W2, b2, W3, b3)


def kernel(user_id, user_age, country, movie_id, movie_genres,
           user_table, age_table, country_table, movie_table, genre_table,
           W1, b1, W2, b2, W3, b3):
    uid = user_id.reshape(B)
    aid = user_age.reshape(B)
    cid = country.reshape(B)
    mid = movie_id.reshape(B)
    e4 = _sc_gather(uid, aid, cid, mid,
                    user_table, age_table, country_table, movie_table)
    gt_pad = jnp.pad(genre_table, ((0, GPAD - GVOC), (0, 0)))
    out = _mlp(e4, movie_genres, gt_pad,
               W1, b1.reshape(1, 256), W2, b2.reshape(1, 64),
               W3, b3.reshape(1, 1))
    return out.reshape(B, 1, 1)


# speedup vs baseline: 17.7081x; 1.3401x over previous
import functools

import jax
import jax.numpy as jnp
from jax import lax
from jax.experimental import pallas as pl
from jax.experimental.pallas import tpu as pltpu
from jax.experimental.pallas import tpu_sc as plsc

B = 16384
D = 32
NC = 2
NS = 16
NW = NC * NS
B_PER_W = B // NW
CHUNK = 128
N_CHUNKS = B_PER_W // CHUNK
N_GROUPS = B_PER_W // 16

GVOC = 21
GPAD = 32
N_GENRES = 19

BLK = 4096


def _sc_gather_body(uid, aid, cid, mid, gen, ut, at_, ct, mt,
                    out, cnt_out,
                    iu, ia, ic, im, gv, ru, ra, rc, rm, cnt, fbv, idxb,
                    onev, csh, isem, gsem, ssem, wsem):
    e4 = (ru, ra, rc, rm)
    sid = lax.axis_index("s")
    wid = sid * NC + lax.axis_index("c")
    base = wid * B_PER_W
    creg = sid * (B_PER_W * D)
    feats = ((uid, ut, iu), (aid, at_, ia), (cid, ct, ic), (mid, mt, im))
    loads = [pltpu.async_copy(idx.at[pl.ds(base, B_PER_W)], iv, isem)
             for idx, _, iv in feats]
    loads.append(pltpu.async_copy(gen.at[:, pl.ds(base, B_PER_W)], gv, isem))
    for h in loads:
        h.wait()
    gathers = []
    for (_, tab, iv), rv in zip(feats, e4):
        for c in range(N_CHUNKS):
            gathers.append(pltpu.async_copy(
                tab.at[iv.at[pl.ds(c * CHUNK, CHUNK)]],
                rv.at[pl.ds(c * CHUNK, CHUNK)], gsem))

    zeros = jnp.zeros((16,), jnp.float32)
    ones = jnp.ones((16,), jnp.float32)
    lane = lax.iota(jnp.int32, 16)

    @pl.loop(0, B_PER_W * D // 16)
    def _zero(i):
        cnt[pl.ds(i * 16, 16)] = zeros

    for k in range(CHUNK // 16):
        onev[pl.ds(k * 16, 16)] = ones

    pltpu.sync_copy(cnt, csh.at[pl.ds(creg, B_PER_W * D)])

    @pl.loop(0, N_GROUPS)
    def _fbase(g):
        bvec = g * 16 + lane
        fbv[pl.ds(g * 16, 16)] = (creg
                                  + lax.shift_right_logical(bvec, 2) * (4 * D)
                                  + (bvec & 3) * D)

    @pl.loop(0, N_GENRES)
    def _build(t):
        for c in range(N_CHUNKS):
            for k in range(CHUNK // 16):
                o = c * CHUNK + k * 16
                idxb[t * N_CHUNKS + c, pl.ds(k * 16, 16)] = (
                    fbv[pl.ds(o, 16)] + gv[t, pl.ds(o, 16)])

    scats = [pltpu.async_copy(onev, csh.at[idxb.at[r]], ssem, add=True)
             for r in range(N_GENRES * N_CHUNKS)]

    for h in scats:
        h.wait()
    pltpu.sync_copy(csh.at[pl.ds(creg, B_PER_W * D)], cnt)
    for h in gathers:
        h.wait()
    writes = [pltpu.async_copy(cnt, cnt_out.at[pl.ds(wid * (B_PER_W * D),
                                                     B_PER_W * D)], wsem)]
    writes += [pltpu.async_copy(rv, out.at[pl.ds(base, B_PER_W),
                                           pl.ds(f * D, D)], wsem)
               for f, rv in enumerate(e4)]
    for h in writes:
        h.wait()


def _sc_gather(uid, aid, cid, mid, gen, ut, at_, ct, mt):
    mesh = plsc.VectorSubcoreMesh(core_axis_name="c", subcore_axis_name="s")
    f = functools.partial(
        pl.kernel,
        mesh=mesh,
        compiler_params=pltpu.CompilerParams(use_tc_tiling_on_sc=False),
        out_type=[jax.ShapeDtypeStruct((B, 4 * D), jnp.float32),
                  jax.ShapeDtypeStruct((B * D,), jnp.float32)],
        scratch_types=[pltpu.VMEM((B_PER_W,), jnp.int32)] * 4
                      + [pltpu.VMEM((N_GENRES, B_PER_W), jnp.int32)]
                      + [pltpu.VMEM((B_PER_W, D), jnp.float32)] * 4
                      + [pltpu.VMEM((B_PER_W * D,), jnp.float32),
                         pltpu.VMEM((B_PER_W,), jnp.int32),
                         pltpu.VMEM((N_GENRES * N_CHUNKS, CHUNK), jnp.int32),
                         pltpu.VMEM((CHUNK,), jnp.float32),
                         pltpu.VMEM_SHARED((NS * B_PER_W * D,), jnp.float32)]
                      + [pltpu.SemaphoreType.DMA] * 4,
    )(_sc_gather_body)
    return f(uid, aid, cid, mid, gen, ut, at_, ct, mt)


def _mlp_body(e4, cntp, gt, W1, b1, W2, b2, W3, b3, out):
    w1 = W1[...]
    h = jnp.dot(e4[...], w1[0:4 * D, :], preferred_element_type=jnp.float32)
    cp = cntp[...]
    counts = jnp.stack([cp[:, k * GPAD:(k + 1) * GPAD] for k in range(4)],
                       axis=1).reshape(BLK, GPAD)
    gm = jnp.dot(gt[...] * jnp.float32(1.0 / N_GENRES), w1[4 * D:5 * D, :],
                 preferred_element_type=jnp.float32)
    h += jnp.dot(counts, gm, preferred_element_type=jnp.float32)
    h = jnp.maximum(h + b1[...], 0.0)
    h2 = jnp.maximum(jnp.dot(h, W2[...], preferred_element_type=jnp.float32) + b2[...], 0.0)
    out[...] = jnp.dot(h2, W3[...], preferred_element_type=jnp.float32) + b3[...]


def _mlp(e4, cntp, gt, W1, b1, W2, b2, W3, b3):
    grid = (B // BLK,)
    bspec = lambda shape: pl.BlockSpec(shape, lambda i: (i, 0))
    full = lambda shape: pl.BlockSpec(shape, lambda i: (0, 0))
    return pl.pallas_call(
        _mlp_body,
        grid=grid,
        in_specs=[
            bspec((BLK, 4 * D)),
            bspec((BLK // 4, 4 * GPAD)),
            full((GPAD, D)),
            full((5 * D, 256)), full((1, 256)),
            full((256, 64)), full((1, 64)),
            full((64, 1)), full((1, 1)),
        ],
        out_specs=bspec((BLK, 1)),
        out_shape=jax.ShapeDtypeStruct((B, 1), jnp.float32),
    )(e4, cntp, gt, W1, b1, W2, b2, W3, b3)


def kernel(user_id, user_age, country, movie_id, movie_genres,
           user_table, age_table, country_table, movie_table, genre_table,
           W1, b1, W2, b2, W3, b3):
    uid = user_id.reshape(B)
    aid = user_age.reshape(B)
    cid = country.reshape(B)
    mid = movie_id.reshape(B)
    e4, cntp = _sc_gather(uid, aid, cid, mid, movie_genres.T,
                          user_table, age_table, country_table, movie_table)
    cntp = cntp.reshape(B // 4, 4 * GPAD)
    gt_pad = jnp.pad(genre_table, ((0, GPAD - GVOC), (0, 0)))
    out = _mlp(e4, cntp, gt_pad,
               W1, b1.reshape(1, 256), W2, b2.reshape(1, 64),
               W3, b3.reshape(1, 1))
    return out.reshape(B, 1, 1)


# speedup vs baseline: 27.7659x; 1.5680x over previous
import functools

import jax
import jax.numpy as jnp
from jax import lax
from jax.experimental import pallas as pl
from jax.experimental.pallas import tpu as pltpu
from jax.experimental.pallas import tpu_sc as plsc

B = 16384
D = 32
NC = 2
NS = 16
NW = NC * NS
B_PER_W = B // NW
CHUNK = 128
N_CHUNKS = B_PER_W // CHUNK
N_GROUPS = B_PER_W // 16

GVOC = 21
GPAD = 32
N_GENRES = 19

BLK = 4096


def _sc_gather_body(uid, aid, cid, mid, gen, ut, at_, ct, mt,
                    out, cnt_out,
                    iu, ia, ic, im, gv, ru, ra, rc, rm, cnt, fbv, idxb,
                    onev, csh, uts, ats, cts, mts, isem, gsem, ssem, wsem):
    e4 = (ru, ra, rc, rm)
    sid = lax.axis_index("s")
    wid = sid * NC + lax.axis_index("c")
    base = wid * B_PER_W
    creg = sid * (B_PER_W * D)
    feats = ((uid, uts, iu), (aid, ats, ia), (cid, cts, ic), (mid, mts, im))
    loads = [pltpu.async_copy(idx.at[pl.ds(base, B_PER_W)], iv, isem)
             for idx, _, iv in feats]
    loads.append(pltpu.async_copy(gen.at[:, pl.ds(base, B_PER_W)], gv, isem))
    @pl.when(sid == 0)
    def _stage_tables():
        pltpu.sync_copy(ut, uts)
        pltpu.sync_copy(at_, ats)
        pltpu.sync_copy(ct, cts)
        pltpu.sync_copy(mt, mts)

    for h in loads:
        h.wait()
    plsc.subcore_barrier()
    gathers = []
    for (_, tab, iv), rv in zip(feats, e4):
        for c in range(N_CHUNKS):
            gathers.append(pltpu.async_copy(
                tab.at[iv.at[pl.ds(c * CHUNK, CHUNK)]],
                rv.at[pl.ds(c * CHUNK, CHUNK)], gsem))

    zeros = jnp.zeros((16,), jnp.float32)
    ones = jnp.ones((16,), jnp.float32)
    lane = lax.iota(jnp.int32, 16)

    @pl.loop(0, B_PER_W * D // 16)
    def _zero(i):
        cnt[pl.ds(i * 16, 16)] = zeros

    for k in range(CHUNK // 16):
        onev[pl.ds(k * 16, 16)] = ones

    pltpu.sync_copy(cnt, csh.at[pl.ds(creg, B_PER_W * D)])

    @pl.loop(0, N_GROUPS)
    def _fbase(g):
        bvec = g * 16 + lane
        fbv[pl.ds(g * 16, 16)] = (creg
                                  + lax.shift_right_logical(bvec, 2) * (4 * D)
                                  + (bvec & 3) * D)

    @pl.loop(0, N_GENRES)
    def _build(t):
        for c in range(N_CHUNKS):
            for k in range(CHUNK // 16):
                o = c * CHUNK + k * 16
                idxb[t * N_CHUNKS + c, pl.ds(k * 16, 16)] = (
                    fbv[pl.ds(o, 16)] + gv[t, pl.ds(o, 16)])

    scats = [pltpu.async_copy(onev, csh.at[idxb.at[r]], ssem, add=True)
             for r in range(N_GENRES * N_CHUNKS)]

    for h in scats:
        h.wait()
    pltpu.sync_copy(csh.at[pl.ds(creg, B_PER_W * D)], cnt)
    for h in gathers:
        h.wait()
    writes = [pltpu.async_copy(cnt, cnt_out.at[pl.ds(wid * (B_PER_W * D),
                                                     B_PER_W * D)], wsem)]
    writes += [pltpu.async_copy(rv, out.at[pl.ds(base, B_PER_W),
                                           pl.ds(f * D, D)], wsem)
               for f, rv in enumerate(e4)]
    for h in writes:
        h.wait()


def _sc_gather(uid, aid, cid, mid, gen, ut, at_, ct, mt):
    mesh = plsc.VectorSubcoreMesh(core_axis_name="c", subcore_axis_name="s")
    f = functools.partial(
        pl.kernel,
        mesh=mesh,
        compiler_params=pltpu.CompilerParams(use_tc_tiling_on_sc=False),
        out_type=[jax.ShapeDtypeStruct((B, 4 * D), jnp.float32),
                  jax.ShapeDtypeStruct((B * D,), jnp.float32)],
        scratch_types=[pltpu.VMEM((B_PER_W,), jnp.int32)] * 4
                      + [pltpu.VMEM((N_GENRES, B_PER_W), jnp.int32)]
                      + [pltpu.VMEM((B_PER_W, D), jnp.float32)] * 4
                      + [pltpu.VMEM((B_PER_W * D,), jnp.float32),
                         pltpu.VMEM((B_PER_W,), jnp.int32),
                         pltpu.VMEM((N_GENRES * N_CHUNKS, CHUNK), jnp.int32),
                         pltpu.VMEM((CHUNK,), jnp.float32),
                         pltpu.VMEM_SHARED((NS * B_PER_W * D,), jnp.float32),
                         pltpu.VMEM_SHARED((944, D), jnp.float32),
                         pltpu.VMEM_SHARED((128, D), jnp.float32),
                         pltpu.VMEM_SHARED((16, D), jnp.float32),
                         pltpu.VMEM_SHARED((1683, D), jnp.float32)]
                      + [pltpu.SemaphoreType.DMA] * 4,
    )(_sc_gather_body)
    return f(uid, aid, cid, mid, gen, ut, at_, ct, mt)


def _mlp_body(e4, cntp, gt, W1, b1, W2, b2, W3, b3, out):
    w1 = W1[...]
    h = jnp.dot(e4[...], w1[0:4 * D, :], preferred_element_type=jnp.float32)
    cp = cntp[...]
    counts = jnp.stack([cp[:, k * GPAD:(k + 1) * GPAD] for k in range(4)],
                       axis=1).reshape(BLK, GPAD)
    gm = jnp.dot(gt[...] * jnp.float32(1.0 / N_GENRES), w1[4 * D:5 * D, :],
                 preferred_element_type=jnp.float32)
    h += jnp.dot(counts, gm, preferred_element_type=jnp.float32)
    h = jnp.maximum(h + b1[...], 0.0)
    h2 = jnp.maximum(jnp.dot(h, W2[...], preferred_element_type=jnp.float32) + b2[...], 0.0)
    o = lax.dot_general(W3[...], h2, (((0,), (1,)), ((), ())),
                        preferred_element_type=jnp.float32)
    out[...] = o[0] + b3[0, 0]


def _mlp(e4, cntp, gt, W1, b1, W2, b2, W3, b3):
    grid = (B // BLK,)
    bspec = lambda shape: pl.BlockSpec(shape, lambda i: (i, 0))
    full = lambda shape: pl.BlockSpec(shape, lambda i: (0, 0))
    return pl.pallas_call(
        _mlp_body,
        grid=grid,
        in_specs=[
            bspec((BLK, 4 * D)),
            bspec((BLK // 4, 4 * GPAD)),
            full((GPAD, D)),
            full((5 * D, 256)), full((1, 256)),
            full((256, 64)), full((1, 64)),
            full((64, 1)), full((1, 1)),
        ],
        out_specs=pl.BlockSpec((BLK,), lambda i: (i,)),
        out_shape=jax.ShapeDtypeStruct((B,), jnp.float32),
    )(e4, cntp, gt, W1, b1, W2, b2, W3, b3)


def kernel(user_id, user_age, country, movie_id, movie_genres,
           user_table, age_table, country_table, movie_table, genre_table,
           W1, b1, W2, b2, W3, b3):
    uid = user_id.reshape(B)
    aid = user_age.reshape(B)
    cid = country.reshape(B)
    mid = movie_id.reshape(B)
    e4, cntp = _sc_gather(uid, aid, cid, mid, movie_genres.T,
                          user_table, age_table, country_table, movie_table)
    cntp = cntp.reshape(B // 4, 4 * GPAD)
    gt_pad = jnp.pad(genre_table, ((0, GPAD - GVOC), (0, 0)))
    out = _mlp(e4, cntp, gt_pad,
               W1, b1.reshape(1, 256), W2, b2.reshape(1, 64),
               W3, b3.reshape(1, 1))
    return out.reshape(B, 1, 1)
